# baseline (speedup 1.0000x reference)
import jax
import jax.numpy as jnp
from jax import lax
from jax.experimental import pallas as pl
from jax.experimental.pallas import tpu as pltpu
from jax.experimental.pallas import tpu_sc as plsc

N = 100000
D = 128
C = 10
CP = 16
S = 512
DS = S
SA = S + 8
NC = 2
NS = 16
NH = 2
GRIDH = 2
TPW = 25088
NPH = GRIDH * TPW
NP = NH * NPH
RQ = TPW // 8
PRH = GRIDH * RQ
WR = PRH // 4
L = 16
ZR = 40
ST = S // NS


def _mlp_body(x_ref, w1_ref, b1_ref, w2_ref, b2_ref, out_ref):
    h = jnp.dot(x_ref[...], w1_ref[...], preferred_element_type=jnp.float32)
    h = jnp.maximum(h + b1_ref[...], 0.0)
    w2 = jnp.pad(w2_ref[...], ((0, 0), (0, CP - C)))
    b2 = jnp.pad(b2_ref[...], ((0, 0), (0, CP - C)))
    logits = jnp.dot(h, w2, preferred_element_type=jnp.float32) + b2
    out_ref[...] = jnp.concatenate(
        [logits[k * RQ:(k + 1) * RQ, :] for k in range(8)], axis=1)


def _mlp(x, W1, b1, W2, b2, half):
    return pl.pallas_call(
        _mlp_body,
        grid=(GRIDH,),
        in_specs=[
            pl.BlockSpec((TPW, D), lambda i: (i + half * GRIDH, 0)),
            pl.BlockSpec((D, D), lambda i: (0, 0)),
            pl.BlockSpec((1, D), lambda i: (0, 0)),
            pl.BlockSpec((D, C), lambda i: (0, 0)),
            pl.BlockSpec((1, C), lambda i: (0, 0)),
        ],
        out_specs=pl.BlockSpec((RQ, D), lambda i: (i, 0)),
        out_shape=jax.ShapeDtypeStruct((PRH, D), jnp.float32),
    )(x, W1, b1, W2, b2)


LASTB = N - (NPH + TPW + 7 * RQ + WR)
TAILB = WR - LASTB


def _make_seg_sum(half):
    def _seg_body(log_hbm, bat_hbm, zero_hbm, out_hbm,
                  log_v, idx_v, red_v, wbuf, shacc, sem0, sem1, sem2):
        cid = lax.axis_index("c")
        sid = lax.axis_index("s")
        wid = cid * NS + sid
        j = wid // 4
        q = wid % 4
        rbase = sid * SA

        cp_log = pltpu.async_copy(
            log_hbm.at[pl.ds(q * WR, WR), pl.ds(j * CP, CP)], log_v, sem0)
        bbase = half * NPH + (q // 2) * TPW + j * RQ + (q % 2) * WR
        if half == 0:
            cp_idx = pltpu.async_copy(bat_hbm.at[pl.ds(bbase, WR)], idx_v,
                                      sem1)
        else:
            cp_idx = pltpu.async_copy(bat_hbm.at[pl.ds(bbase, LASTB)],
                                      idx_v.at[pl.ds(0, LASTB)], sem1)
            last = wid == NW - 1

            @pl.when(jnp.logical_not(last))
            def _rest():
                pltpu.async_copy(bat_hbm.at[pl.ds(bbase + LASTB, TAILB)],
                                 idx_v.at[pl.ds(LASTB, TAILB)], sem2).wait()

            @pl.when(last)
            def _fill():
                dump = jnp.full((L,), DS, jnp.int32)
                for t in range(TAILB // L):
                    idx_v[pl.ds(LASTB + t * L, L)] = dump

        zcp = pltpu.async_copy(zero_hbm, shacc.at[pl.ds(rbase, SA), :], sem2)

        cp_idx.wait()
        bias = jnp.full((L,), 1, jnp.int32) * rbase
        for t in range(WR // L):
            idx_v[pl.ds(t * L, L)] = idx_v[pl.ds(t * L, L)] + bias
        zcp.wait()
        cp_log.wait()

        pltpu.async_copy(log_v, shacc.at[idx_v], sem1, add=True).wait()

        plsc.subcore_barrier()
        rcps = [
            pltpu.async_copy(
                shacc.at[pl.ds(k * SA + sid * ST, ST), :], red_v.at[k], sem2)
            for k in range(NS)
        ]
        for cp in rcps:
            cp.wait()
        for r in range(ST):
            v = red_v[0, r, :]
            for k in range(1, NS):
                v = v + red_v[k, r, :]
            wbuf[r, :] = v
        pltpu.sync_copy(wbuf, out_hbm.at[cid, pl.ds(sid * ST, ST), :])

    return pl.kernel(
        _seg_body,
        out_type=jax.ShapeDtypeStruct((NC, S, CP), jnp.float32),
        mesh=plsc.VectorSubcoreMesh(core_axis_name="c", subcore_axis_name="s"),
        compiler_params=pltpu.CompilerParams(use_tc_tiling_on_sc=False),
        scratch_types=[
            pltpu.VMEM((WR, CP), jnp.float32),
            pltpu.VMEM((WR,), jnp.int32),
            pltpu.VMEM((NS, ST, CP), jnp.float32),
            pltpu.VMEM((ST, CP), jnp.float32),
            pltpu.VMEM_SHARED((NS * SA, CP), jnp.float32),
            pltpu.SemaphoreType.DMA,
            pltpu.SemaphoreType.DMA,
            pltpu.SemaphoreType.DMA,
        ],
    )


NW = NC * NS
_seg_sum_a = _make_seg_sum(0)
_seg_sum_b = _make_seg_sum(1)


def _combine_body(pa_ref, pb_ref, out_ref):
    out_ref[...] = (pa_ref[0] + pa_ref[1]) + (pb_ref[0] + pb_ref[1])


def _combine(pa, pb):
    return pl.pallas_call(
        _combine_body,
        out_shape=jax.ShapeDtypeStruct((S, CP), jnp.float32),
    )(pa, pb)


@jax.jit
def _run(x, batch, W1, b1, W2, b2):
    bat = batch.astype(jnp.int32)
    zero = jnp.zeros((SA, CP), jnp.float32)
    b1r, b2r = b1[None, :], b2[None, :]
    logits_a = _mlp(x, W1, b1r, W2, b2r, 0)
    pa = _seg_sum_a(logits_a, bat, zero)
    logits_b = _mlp(x, W1, b1r, W2, b2r, 1)
    pb = _seg_sum_b(logits_b, bat, zero)
    out = _combine(pa, pb)
    return out[:, :C]


def kernel(x, batch, W1, b1, W2, b2):
    return _run(x, batch, W1, b1, W2, b2)


# speedup vs baseline: 1.0751x; 1.0751x over previous
import jax
import jax.numpy as jnp
from jax import lax
from jax.experimental import pallas as pl
from jax.experimental.pallas import tpu as pltpu
from jax.experimental.pallas import tpu_sc as plsc

N = 100000
D = 128
C = 10
CP = 16
S = 512
DS = S
SA = S + 8
NC = 2
NS = 16
GRID = 4
TPW = 25088
NP = GRID * TPW
RQ = TPW // 8
PR = GRID * RQ
WROWS = PR // 4


def _mlp_body(x_ref, w1_ref, b1_ref, w2_ref, b2_ref, out_ref):
    h = jnp.dot(x_ref[...], w1_ref[...], preferred_element_type=jnp.float32)
    h = jnp.maximum(h + b1_ref[...], 0.0)
    logits = (
        jnp.dot(h, w2_ref[...], preferred_element_type=jnp.float32) + b2_ref[...]
    )
    out_ref[...] = jnp.concatenate(
        [logits[k * RQ:(k + 1) * RQ, :] for k in range(8)], axis=1)


def _mlp(x, W1, b1, W2p, b2p):
    return pl.pallas_call(
        _mlp_body,
        grid=(GRID,),
        in_specs=[
            pl.BlockSpec((TPW, D), lambda i: (i, 0)),
            pl.BlockSpec((D, D), lambda i: (0, 0)),
            pl.BlockSpec((1, D), lambda i: (0, 0)),
            pl.BlockSpec((D, CP), lambda i: (0, 0)),
            pl.BlockSpec((1, CP), lambda i: (0, 0)),
        ],
        out_specs=pl.BlockSpec((RQ, D), lambda i: (i, 0)),
        out_shape=jax.ShapeDtypeStruct((PR, D), jnp.float32),
    )(x, W1, b1, W2p, b2p)


def _seg_body(log_hbm, bat_hbm, zero_hbm, out_hbm,
              log_v, idx_v, shacc, sem0, sem1):
    cid = lax.axis_index("c")
    sid = lax.axis_index("s")
    wid = cid * NS + sid
    j = wid // 4
    q = wid % 4
    rbase = q * WROWS

    cp_log = pltpu.async_copy(
        log_hbm.at[pl.ds(rbase, WROWS), pl.ds(j * CP, CP)], log_v, sem0)
    cp_idx = pltpu.async_copy(
        bat_hbm.at[pl.ds(q * TPW + j * RQ, WROWS)], idx_v, sem1)

    @pl.when(sid == 0)
    def _zero():
        pltpu.sync_copy(zero_hbm, shacc)

    cp_idx.wait()
    plsc.subcore_barrier()
    cp_log.wait()

    pltpu.async_copy(log_v, shacc.at[idx_v], sem1, add=True).wait()

    plsc.subcore_barrier()
    st = S // NS
    pltpu.sync_copy(shacc.at[pl.ds(sid * st, st), :],
                    out_hbm.at[cid, pl.ds(sid * st, st), :])


_seg_sum = pl.kernel(
    _seg_body,
    out_type=jax.ShapeDtypeStruct((NC, S, CP), jnp.float32),
    mesh=plsc.VectorSubcoreMesh(core_axis_name="c", subcore_axis_name="s"),
    compiler_params=pltpu.CompilerParams(use_tc_tiling_on_sc=False),
    scratch_types=[
        pltpu.VMEM((WROWS, CP), jnp.float32),
        pltpu.VMEM((WROWS,), jnp.int32),
        pltpu.VMEM_SHARED((SA, CP), jnp.float32),
        pltpu.SemaphoreType.DMA,
        pltpu.SemaphoreType.DMA,
    ],
)


def _combine_body(p_ref, out_ref):
    out_ref[...] = p_ref[0] + p_ref[1]


def _combine(p):
    return pl.pallas_call(
        _combine_body,
        out_shape=jax.ShapeDtypeStruct((S, CP), jnp.float32),
    )(p)


@jax.jit
def _run(x, batch, W1, b1, W2, b2):
    W2p = jnp.zeros((D, CP), W2.dtype).at[:, :C].set(W2)
    b2p = jnp.zeros((CP,), b2.dtype).at[:C].set(b2)
    logits = _mlp(x, W1, b1[None, :], W2p, b2p[None, :])
    bat = jnp.concatenate(
        [batch.astype(jnp.int32), jnp.full((NP - N,), DS, jnp.int32)])
    zero = jnp.zeros((SA, CP), jnp.float32)
    partials = _seg_sum(logits, bat, zero)
    out = _combine(partials)
    return out[:, :C]


def kernel(x, batch, W1, b1, W2, b2):
    return _run(x, batch, W1, b1, W2, b2)


# speedup vs baseline: 1.1213x; 1.0430x over previous
import jax
import jax.numpy as jnp
from jax import lax
from jax.experimental import pallas as pl
from jax.experimental.pallas import tpu as pltpu
from jax.experimental.pallas import tpu_sc as plsc

N = 100000
D = 128
C = 10
CP = 16
S = 512
DS = S
SA = S + 8
NC = 2
NS = 16
GRID = 4
TPW = 25088
NP = GRID * TPW
RQ = TPW // 8
PR = GRID * RQ
WROWS = PR // 4


def _mlp_body(x_ref, w1_ref, b1_ref, w2_ref, b2_ref, out_ref):
    h = jnp.dot(x_ref[...], w1_ref[...], preferred_element_type=jnp.float32)
    h = jnp.maximum(h + b1_ref[...], 0.0)
    w2 = jnp.pad(w2_ref[...], ((0, 0), (0, CP - C)))
    b2 = jnp.pad(b2_ref[...], ((0, 0), (0, CP - C)))
    logits = jnp.dot(h, w2, preferred_element_type=jnp.float32) + b2
    out_ref[...] = jnp.concatenate(
        [logits[k * RQ:(k + 1) * RQ, :] for k in range(8)], axis=1)


def _mlp(x, W1, b1, W2, b2):
    return pl.pallas_call(
        _mlp_body,
        grid=(GRID,),
        in_specs=[
            pl.BlockSpec((TPW, D), lambda i: (i, 0)),
            pl.BlockSpec((D, D), lambda i: (0, 0)),
            pl.BlockSpec((1, D), lambda i: (0, 0)),
            pl.BlockSpec((D, C), lambda i: (0, 0)),
            pl.BlockSpec((1, C), lambda i: (0, 0)),
        ],
        out_specs=pl.BlockSpec((RQ, D), lambda i: (i, 0)),
        out_shape=jax.ShapeDtypeStruct((PR, D), jnp.float32),
    )(x, W1, b1, W2, b2)


def _seg_body(log_hbm, bat_hbm, zero_hbm, out_hbm,
              log_v, idx_v, shacc, sem0, sem1):
    cid = lax.axis_index("c")
    sid = lax.axis_index("s")
    wid = cid * NS + sid
    j = wid // 4
    q = wid % 4
    rbase = q * WROWS

    cp_log = pltpu.async_copy(
        log_hbm.at[pl.ds(rbase, WROWS), pl.ds(j * CP, CP)], log_v, sem0)
    cp_idx = pltpu.async_copy(
        bat_hbm.at[pl.ds(q * TPW + j * RQ, WROWS)], idx_v, sem1)

    @pl.when(sid == 0)
    def _zero():
        pltpu.sync_copy(zero_hbm, shacc)

    cp_idx.wait()
    plsc.subcore_barrier()
    cp_log.wait()

    pltpu.async_copy(log_v, shacc.at[idx_v], sem1, add=True).wait()

    plsc.subcore_barrier()
    st = S // NS
    pltpu.sync_copy(shacc.at[pl.ds(sid * st, st), :],
                    out_hbm.at[cid, pl.ds(sid * st, st), :])


_seg_sum = pl.kernel(
    _seg_body,
    out_type=jax.ShapeDtypeStruct((NC, S, CP), jnp.float32),
    mesh=plsc.VectorSubcoreMesh(core_axis_name="c", subcore_axis_name="s"),
    compiler_params=pltpu.CompilerParams(use_tc_tiling_on_sc=False),
    scratch_types=[
        pltpu.VMEM((WROWS, CP), jnp.float32),
        pltpu.VMEM((WROWS,), jnp.int32),
        pltpu.VMEM_SHARED((SA, CP), jnp.float32),
        pltpu.SemaphoreType.DMA,
        pltpu.SemaphoreType.DMA,
    ],
)


def _combine_body(p_ref, out_ref):
    out_ref[...] = p_ref[0] + p_ref[1]


def _combine(p):
    return pl.pallas_call(
        _combine_body,
        out_shape=jax.ShapeDtypeStruct((S, CP), jnp.float32),
    )(p)


@jax.jit
def _run(x, batch, W1, b1, W2, b2):
    logits = _mlp(x, W1, b1[None, :], W2, b2[None, :])
    bat = jnp.concatenate(
        [batch.astype(jnp.int32), jnp.full((NP - N,), DS, jnp.int32)])
    zero = jnp.zeros((SA, CP), jnp.float32)
    partials = _seg_sum(logits, bat, zero)
    out = _combine(partials)
    return out[:, :C]


def kernel(x, batch, W1, b1, W2, b2):
    return _run(x, batch, W1, b1, W2, b2)


# speedup vs baseline: 1.1403x; 1.0169x over previous
import jax
import jax.numpy as jnp
from jax import lax
from jax.experimental import pallas as pl
from jax.experimental.pallas import tpu as pltpu
from jax.experimental.pallas import tpu_sc as plsc

N = 100000
D = 128
C = 10
CP = 16
S = 512
DS = S
SA = S + 8
NC = 2
NS = 16
GRID = 4
TPW = 25088
NP = GRID * TPW
RQ = TPW // 8
PR = GRID * RQ
WROWS = PR // 4
TAIL = NP - N
HEAD = WROWS - TAIL


def _mlp_body(x_ref, w1_ref, b1_ref, w2_ref, b2_ref, out_ref):
    h = jnp.dot(x_ref[...], w1_ref[...], preferred_element_type=jnp.float32)
    h = jnp.maximum(h + b1_ref[...], 0.0)
    w2 = jnp.pad(w2_ref[...], ((0, 0), (0, CP - C)))
    b2 = jnp.pad(b2_ref[...], ((0, 0), (0, CP - C)))
    logits = jnp.dot(h, w2, preferred_element_type=jnp.float32) + b2
    out_ref[...] = jnp.concatenate(
        [logits[k * RQ:(k + 1) * RQ, :] for k in range(8)], axis=1)


def _mlp(x, W1, b1, W2, b2):
    return pl.pallas_call(
        _mlp_body,
        grid=(GRID,),
        in_specs=[
            pl.BlockSpec((TPW, D), lambda i: (i, 0)),
            pl.BlockSpec((D, D), lambda i: (0, 0)),
            pl.BlockSpec((1, D), lambda i: (0, 0)),
            pl.BlockSpec((D, C), lambda i: (0, 0)),
            pl.BlockSpec((1, C), lambda i: (0, 0)),
        ],
        out_specs=pl.BlockSpec((RQ, D), lambda i: (i, 0)),
        out_shape=jax.ShapeDtypeStruct((PR, D), jnp.float32),
    )(x, W1, b1, W2, b2)


def _seg_body(log_hbm, bat_hbm, pad_hbm, zero_hbm, out_hbm,
              log_v, idx_v, shacc, sem0, sem1, sem2):
    cid = lax.axis_index("c")
    sid = lax.axis_index("s")
    wid = cid * NS + sid
    j = wid // 4
    q = wid % 4
    rbase = q * WROWS

    cp_log = pltpu.async_copy(
        log_hbm.at[pl.ds(rbase, WROWS), pl.ds(j * CP, CP)], log_v, sem0)
    bbase = q * TPW + j * RQ
    cp_idx = pltpu.async_copy(
        bat_hbm.at[pl.ds(bbase, HEAD)], idx_v.at[pl.ds(0, HEAD)], sem1)
    last = wid == NC * NS - 1

    @pl.when(jnp.logical_not(last))
    def _tail_ids():
        pltpu.async_copy(bat_hbm.at[pl.ds(bbase + HEAD, TAIL)],
                         idx_v.at[pl.ds(HEAD, TAIL)], sem2).wait()

    @pl.when(last)
    def _tail_pad():
        pltpu.async_copy(pad_hbm, idx_v.at[pl.ds(HEAD, TAIL)], sem2).wait()

    @pl.when(sid == 0)
    def _zero():
        pltpu.sync_copy(zero_hbm, shacc)

    cp_idx.wait()
    plsc.subcore_barrier()
    cp_log.wait()

    pltpu.async_copy(log_v, shacc.at[idx_v], sem1, add=True).wait()

    plsc.subcore_barrier()
    st = S // NS
    pltpu.sync_copy(shacc.at[pl.ds(sid * st, st), :],
                    out_hbm.at[cid, pl.ds(sid * st, st), :])


_seg_sum = pl.kernel(
    _seg_body,
    out_type=jax.ShapeDtypeStruct((NC, S, CP), jnp.float32),
    mesh=plsc.VectorSubcoreMesh(core_axis_name="c", subcore_axis_name="s"),
    compiler_params=pltpu.CompilerParams(use_tc_tiling_on_sc=False),
    scratch_types=[
        pltpu.VMEM((WROWS, CP), jnp.float32),
        pltpu.VMEM((WROWS,), jnp.int32),
        pltpu.VMEM_SHARED((SA, CP), jnp.float32),
        pltpu.SemaphoreType.DMA,
        pltpu.SemaphoreType.DMA,
        pltpu.SemaphoreType.DMA,
    ],
)


def _combine_body(p_ref, out_ref):
    out_ref[...] = p_ref[0] + p_ref[1]


def _combine(p):
    return pl.pallas_call(
        _combine_body,
        out_shape=jax.ShapeDtypeStruct((S, CP), jnp.float32),
    )(p)


@jax.jit
def _run(x, batch, W1, b1, W2, b2):
    logits = _mlp(x, W1, b1[None, :], W2, b2[None, :])
    pad_ids = jnp.full((TAIL,), DS, jnp.int32)
    zero = jnp.zeros((SA, CP), jnp.float32)
    partials = _seg_sum(logits, batch.astype(jnp.int32), pad_ids, zero)
    out = _combine(partials)
    return out[:, :C]


def kernel(x, batch, W1, b1, W2, b2):
    return _run(x, batch, W1, b1, W2, b2)
